# baseline (speedup 1.0000x reference)
import functools

import jax
import jax.numpy as jnp
from jax.experimental import pallas as pl
from jax.experimental.pallas import tpu as pltpu
from jax.experimental.pallas import tpu_sc as plsc

_NC = 2
_NS = 16
_L = 16


def _logits_body(x_ref, gw_ref, logits_ref):
    logits_ref[...] = jnp.dot(x_ref[...], gw_ref[...],
                              preferred_element_type=jnp.float32)


def _router_body(logits_hbm, gs_hbm, route_hbm, row_v, gs_v, route_v):
    wid = jax.lax.axis_index("s") * _NC + jax.lax.axis_index("c")
    tpw = 2
    base = wid * tpw
    pltpu.sync_copy(logits_hbm.at[pl.ds(base, tpw)], row_v)
    nch = 64 // _L
    lane = jax.lax.iota(jnp.int32, _L).astype(jnp.float32)
    for t in range(tpw):
        chunks = [row_v[t, pl.ds(j * _L, _L)] for j in range(nch)]
        m = chunks[0]
        for c in chunks[1:]:
            m = jnp.maximum(m, c)
        m = jnp.max(m)
        ps = [jnp.exp(c - m) for c in chunks]
        s = ps[0]
        for p in ps[1:]:
            s = s + p
        s = jnp.sum(s)
        inv = 1.0 / jnp.broadcast_to(s, (_L,))
        gs = [p * inv for p in ps]
        for j in range(nch):
            gs_v[t, pl.ds(j * _L, _L)] = gs[j]
        v1 = jnp.maximum(jnp.maximum(jnp.max(gs[0]), jnp.max(gs[1])),
                         jnp.maximum(jnp.max(gs[2]), jnp.max(gs[3])))
        big = jnp.float32(64.0)
        i1 = big
        for j in range(nch):
            gidx = lane + jnp.float32(j * _L)
            i1 = jnp.minimum(i1, jnp.min(jnp.where(gs[j] == v1, gidx, big)))
        neg = jnp.float32(-jnp.inf)
        v2 = neg
        for j in range(nch):
            gidx = lane + jnp.float32(j * _L)
            v2 = jnp.maximum(v2, jnp.max(jnp.where(gidx == i1, neg, gs[j])))
        i2 = big
        for j in range(nch):
            gidx = lane + jnp.float32(j * _L)
            gm = jnp.where(gidx == i1, neg, gs[j])
            i2 = jnp.minimum(i2, jnp.min(jnp.where(gm == v2, gidx, big)))
        tw = v1 + v2
        r = jnp.where(lane == 0.0, tw,
                      jnp.where(lane == 1.0, i1,
                                jnp.where(lane == 2.0, i2, 0.0)))
        route_v[t, :] = r
    pltpu.sync_copy(gs_v, gs_hbm.at[pl.ds(base, tpw)])
    pltpu.sync_copy(route_v, route_hbm.at[pl.ds(base, tpw)])


def _experts_body(x_ref, logits_ref, gs_ref, route_ref,
                  w1_ref, b1_ref, w2_ref, b2_ref, w3_ref, b3_ref,
                  out_ref, aux_ref):
    e = pl.program_id(0)

    @pl.when(e == 0)
    def _aux():
        l = logits_ref[...]
        m = jnp.max(l, axis=-1, keepdims=True)
        s = jnp.sum(jnp.exp(l - m), axis=-1, keepdims=True)
        lse = m[:, 0] + jnp.log(s[:, 0])
        z = jnp.mean(lse * lse) * 0.001
        usage = jnp.mean(gs_ref[...], axis=0)
        lbl = -jnp.sum(usage * jnp.log(usage + 1e-9))
        aux_ref[...] = (lbl + z).reshape(1, 1)
        out_ref[...] = jnp.zeros_like(out_ref)

    xx = x_ref[...]
    h1 = jnp.dot(xx, w1_ref[0], preferred_element_type=jnp.float32) + b1_ref[0]
    h2 = jnp.dot(xx, w2_ref[0], preferred_element_type=jnp.float32) + b2_ref[0]
    h = (h1 * jax.nn.sigmoid(h1)) * h2
    eo = jnp.dot(h, w3_ref[0], preferred_element_type=jnp.float32) + b3_ref[0]
    ef = e.astype(jnp.float32)
    rt = route_ref[...]
    w = jnp.where((rt[:, 1:2] == ef) | (rt[:, 2:3] == ef), rt[:, 0:1], 0.0)
    out_ref[...] += eo * w


def kernel(x, gate_w, W1, b1, W2, b2, W3, b3):
    B, S, D = x.shape
    E = gate_w.shape[1]
    H = W1.shape[2]
    T = B * S
    x2 = x.reshape(T, D)
    b1r = b1.reshape(E, 1, H)
    b2r = b2.reshape(E, 1, H)
    b3r = b3.reshape(E, 1, D)

    logits = pl.pallas_call(
        _logits_body,
        in_specs=[pl.BlockSpec((T, D), lambda: (0, 0)),
                  pl.BlockSpec((D, E), lambda: (0, 0))],
        out_specs=pl.BlockSpec((T, E), lambda: (0, 0)),
        out_shape=jax.ShapeDtypeStruct((T, E), jnp.float32),
    )(x2, gate_w)

    router = functools.partial(
        pl.kernel,
        mesh=plsc.VectorSubcoreMesh(core_axis_name="c", subcore_axis_name="s"),
        out_type=[jax.ShapeDtypeStruct((T, E), jnp.float32),
                  jax.ShapeDtypeStruct((T, _L), jnp.float32)],
        scratch_types=[pltpu.VMEM((2, E), jnp.float32),
                       pltpu.VMEM((2, E), jnp.float32),
                       pltpu.VMEM((2, _L), jnp.float32)],
        compiler_params=pltpu.CompilerParams(needs_layout_passes=False),
    )(_router_body)
    gs, route = router(logits)

    out, aux = pl.pallas_call(
        _experts_body,
        grid=(E,),
        in_specs=[
            pl.BlockSpec((T, D), lambda e: (0, 0)),
            pl.BlockSpec((T, E), lambda e: (0, 0)),
            pl.BlockSpec((T, E), lambda e: (0, 0)),
            pl.BlockSpec((T, _L), lambda e: (0, 0)),
            pl.BlockSpec((1, D, H), lambda e: (e, 0, 0)),
            pl.BlockSpec((1, 1, H), lambda e: (e, 0, 0)),
            pl.BlockSpec((1, D, H), lambda e: (e, 0, 0)),
            pl.BlockSpec((1, 1, H), lambda e: (e, 0, 0)),
            pl.BlockSpec((1, H, D), lambda e: (e, 0, 0)),
            pl.BlockSpec((1, 1, D), lambda e: (e, 0, 0)),
        ],
        out_specs=[
            pl.BlockSpec((T, D), lambda e: (0, 0)),
            pl.BlockSpec((1, 1), lambda e: (0, 0)),
        ],
        out_shape=[
            jax.ShapeDtypeStruct((T, D), jnp.float32),
            jax.ShapeDtypeStruct((1, 1), jnp.float32),
        ],
        compiler_params=pltpu.CompilerParams(
            dimension_semantics=("arbitrary",),
        ),
    )(x2, logits, gs, route, W1, b1r, W2, b2r, W3, b3r)
    return out.reshape(B, S, D), aux[0, 0], gs.reshape(B, S, E)
